# baseline (speedup 1.0000x reference)
import functools

import jax
import jax.numpy as jnp
from jax import lax
from jax.experimental import pallas as pl
from jax.experimental.pallas import tpu as pltpu
from jax.experimental.pallas import tpu_sc as plsc

D = 3584
N_IDS = 4096
N_SC = 1536
N_TC = N_IDS - N_SC


NC, NS = 2, 16
NW = NC * NS
BPW = N_SC // NW
CH = 8
NCHUNK = BPW // CH


@functools.partial(
    pl.kernel,
    out_type=jax.ShapeDtypeStruct((N_SC, D), jnp.float32),
    mesh=plsc.VectorSubcoreMesh(core_axis_name="c", subcore_axis_name="s"),
    scratch_types=[
        pltpu.VMEM((BPW,), jnp.int32),
        pltpu.VMEM((2, CH, D), jnp.float32),
        pltpu.SemaphoreType.DMA((2,)),
        pltpu.SemaphoreType.DMA((2,)),
    ],
)
def _sc_call(ids_hbm, table_hbm, out_hbm, idx_v, rows_v, in_sems, out_sems):
    wid = lax.axis_index("s") * NC + lax.axis_index("c")
    base = wid * BPW
    pltpu.sync_copy(ids_hbm.at[pl.ds(base, BPW)], idx_v)

    def gather(c, buf):
        return pltpu.make_async_copy(
            table_hbm.at[idx_v.at[pl.ds(pl.multiple_of(c * CH, 8), CH)]],
            rows_v.at[buf],
            in_sems.at[buf],
        )

    def put(c, buf):
        return pltpu.make_async_copy(
            rows_v.at[buf],
            out_hbm.at[pl.ds(base + c * CH, CH)],
            out_sems.at[buf],
        )

    gather(0, 0).start()
    gather(1, 1).start()
    gather(0, 0).wait()
    put(0, 0).start()

    @pl.loop(0, (NCHUNK - 2) // 2)
    def _(g):
        c1 = 2 * g + 1
        put(c1 - 1, 0).wait()
        gather(c1 + 1, 0).start()
        gather(c1, 1).wait()
        put(c1, 1).start()
        c2 = 2 * g + 2
        put(c2 - 1, 1).wait()
        gather(c2 + 1, 1).start()
        gather(c2, 0).wait()
        put(c2, 0).start()

    cl = NCHUNK - 1
    gather(cl, 1).wait()
    put(cl, 1).start()
    put(cl - 1, 0).wait()
    put(cl, 1).wait()


K = 32
NBUF_TC = 8
LA = 4
NG = N_TC // K


def _tc_body(ids_ref, table_hbm, out_hbm, rows_v, in_sems, out_sems):
    def issue(g, buf):
        for j in range(K):
            idx = ids_ref[g * K + j]
            pltpu.make_async_copy(
                table_hbm.at[pl.ds(idx, 1)],
                rows_v.at[buf, pl.ds(j, 1)],
                in_sems.at[buf],
            ).start()

    def wait_group(buf):
        pltpu.make_async_copy(
            table_hbm.at[pl.ds(0, K)], rows_v.at[buf], in_sems.at[buf]
        ).wait()

    def put(g, buf):
        return pltpu.make_async_copy(
            rows_v.at[buf],
            out_hbm.at[pl.ds(g * K, K)],
            out_sems.at[buf],
        )

    def prologue(g, carry):
        issue(g, g % NBUF_TC)
        return carry

    lax.fori_loop(0, LA, prologue, 0)

    def body(g, carry):
        buf = g % NBUF_TC
        nb = (g + LA) % NBUF_TC

        @pl.when(g + LA < NG)
        def _issue_ahead():
            @pl.when(g >= NBUF_TC - LA)
            def _wait_prev():
                put(g - (NBUF_TC - LA), nb).wait()

            issue(g + LA, nb)

        wait_group(buf)
        put(g, buf).start()
        return carry

    lax.fori_loop(0, NG, body, 0)

    def drain(i, carry):
        g = NG - NBUF_TC + i
        put(g, g % NBUF_TC).wait()
        return carry

    lax.fori_loop(0, NBUF_TC, drain, 0)


_tc_call = pl.pallas_call(
    _tc_body,
    out_shape=jax.ShapeDtypeStruct((N_TC, D), jnp.float32),
    in_specs=[
        pl.BlockSpec(memory_space=pltpu.SMEM),
        pl.BlockSpec(memory_space=pl.ANY),
    ],
    out_specs=pl.BlockSpec(memory_space=pl.ANY),
    scratch_shapes=[
        pltpu.VMEM((NBUF_TC, K, D), jnp.float32),
        pltpu.SemaphoreType.DMA((NBUF_TC,)),
        pltpu.SemaphoreType.DMA((NBUF_TC,)),
    ],
)


def kernel(input_ids, table):
    ids = input_ids.reshape(-1).astype(jnp.int32)
    sc_rows = _sc_call(ids[:N_SC], table)
    tc_rows = _tc_call(ids[N_SC:], table)
    out = jnp.concatenate([sc_rows, tc_rows], axis=0)
    return out.reshape(input_ids.shape + (table.shape[1],))


# speedup vs baseline: 1.3386x; 1.3386x over previous
import functools

import jax
import jax.numpy as jnp
from jax import lax
from jax.experimental import pallas as pl
from jax.experimental.pallas import tpu as pltpu
from jax.experimental.pallas import tpu_sc as plsc

D = 3584
N_IDS = 4096
N_SC = 1024
N_TC = N_IDS - N_SC


NC, NS = 2, 16
NW = NC * NS
BPW = N_SC // NW
CH = 8
NCHUNK = BPW // CH


@functools.partial(
    pl.kernel,
    out_type=jax.ShapeDtypeStruct((N_SC, D), jnp.float32),
    mesh=plsc.VectorSubcoreMesh(core_axis_name="c", subcore_axis_name="s"),
    scratch_types=[
        pltpu.VMEM((BPW,), jnp.int32),
        pltpu.VMEM((2, CH, D), jnp.float32),
        pltpu.SemaphoreType.DMA((2,)),
        pltpu.SemaphoreType.DMA((2,)),
    ],
)
def _sc_call(ids_hbm, table_hbm, out_hbm, idx_v, rows_v, in_sems, out_sems):
    wid = lax.axis_index("s") * NC + lax.axis_index("c")
    base = wid * BPW
    pltpu.sync_copy(ids_hbm.at[pl.ds(base, BPW)], idx_v)

    def gather(c, buf):
        return pltpu.make_async_copy(
            table_hbm.at[idx_v.at[pl.ds(pl.multiple_of(c * CH, 8), CH)]],
            rows_v.at[buf],
            in_sems.at[buf],
        )

    def put(c, buf):
        return pltpu.make_async_copy(
            rows_v.at[buf],
            out_hbm.at[pl.ds(base + c * CH, CH)],
            out_sems.at[buf],
        )

    gather(0, 0).start()
    gather(1, 1).start()
    gather(0, 0).wait()
    put(0, 0).start()

    @pl.loop(0, (NCHUNK - 2) // 2)
    def _(g):
        c1 = 2 * g + 1
        put(c1 - 1, 0).wait()
        gather(c1 + 1, 0).start()
        gather(c1, 1).wait()
        put(c1, 1).start()
        c2 = 2 * g + 2
        put(c2 - 1, 1).wait()
        gather(c2 + 1, 1).start()
        gather(c2, 0).wait()
        put(c2, 0).start()

    cl = NCHUNK - 1
    gather(cl, 1).wait()
    put(cl, 1).start()
    put(cl - 1, 0).wait()
    put(cl, 1).wait()


K = 32
NBUF_TC = 8
LA = 4
NG = N_TC // K


def _tc_body(ids_ref, table_hbm, out_hbm, rows_v, in_sems, out_sems):
    def issue(g, buf):
        for j in range(K):
            idx = ids_ref[g * K + j]
            pltpu.make_async_copy(
                table_hbm.at[pl.ds(idx, 1)],
                rows_v.at[buf, pl.ds(j, 1)],
                in_sems.at[buf],
            ).start()

    def wait_group(buf):
        pltpu.make_async_copy(
            table_hbm.at[pl.ds(0, K)], rows_v.at[buf], in_sems.at[buf]
        ).wait()

    def put(g, buf):
        return pltpu.make_async_copy(
            rows_v.at[buf],
            out_hbm.at[pl.ds(N_SC + g * K, K)],
            out_sems.at[buf],
        )

    def prologue(g, carry):
        issue(g, g % NBUF_TC)
        return carry

    lax.fori_loop(0, LA, prologue, 0)

    def body(g, carry):
        buf = g % NBUF_TC
        nb = (g + LA) % NBUF_TC

        @pl.when(g + LA < NG)
        def _issue_ahead():
            @pl.when(g >= NBUF_TC - LA)
            def _wait_prev():
                put(g - (NBUF_TC - LA), nb).wait()

            issue(g + LA, nb)

        wait_group(buf)
        put(g, buf).start()
        return carry

    lax.fori_loop(0, NG, body, 0)

    def drain(i, carry):
        g = NG - NBUF_TC + i
        put(g, g % NBUF_TC).wait()
        return carry

    lax.fori_loop(0, NBUF_TC, drain, 0)


_tc_call = pl.pallas_call(
    _tc_body,
    out_shape=jax.ShapeDtypeStruct((N_IDS, D), jnp.float32),
    in_specs=[
        pl.BlockSpec(memory_space=pltpu.SMEM),
        pl.BlockSpec(memory_space=pl.ANY),
    ],
    out_specs=pl.BlockSpec(memory_space=pl.ANY),
    scratch_shapes=[
        pltpu.VMEM((NBUF_TC, K, D), jnp.float32),
        pltpu.SemaphoreType.DMA((NBUF_TC,)),
        pltpu.SemaphoreType.DMA((NBUF_TC,)),
    ],
)


def kernel(input_ids, table):
    ids = input_ids.reshape(-1).astype(jnp.int32)
    sc_rows = _sc_call(ids[:N_SC], table)
    tc_full = _tc_call(ids[N_SC:], table)
    out = lax.dynamic_update_slice(tc_full, sc_rows, (0, 0))
    return out.reshape(input_ids.shape + (table.shape[1],))


# speedup vs baseline: 1.5971x; 1.1931x over previous
import functools

import jax
import jax.numpy as jnp
from jax import lax
from jax.experimental import pallas as pl
from jax.experimental.pallas import tpu as pltpu
from jax.experimental.pallas import tpu_sc as plsc

B = 2
S = 2048
D = 3584
NC, NS = 2, 16
NW = NC * NS
BPW = (B * S) // NW
WPR = S // BPW
CH = 16
NCHUNK = BPW // CH


@functools.partial(
    pl.kernel,
    out_type=jax.ShapeDtypeStruct((B, S, D), jnp.float32),
    mesh=plsc.VectorSubcoreMesh(core_axis_name="c", subcore_axis_name="s"),
    scratch_types=[
        pltpu.VMEM((BPW,), jnp.int32),
        pltpu.VMEM((2, CH, D), jnp.float32),
        pltpu.SemaphoreType.DMA((2,)),
        pltpu.SemaphoreType.DMA((2,)),
    ],
)
def _gather_call(ids_hbm, table_hbm, out_hbm, idx_v, rows_v, in_sems, out_sems):
    wid = lax.axis_index("s") * NC + lax.axis_index("c")
    b = wid // WPR
    s0 = (wid % WPR) * BPW
    pltpu.sync_copy(ids_hbm.at[b, pl.ds(s0, BPW)], idx_v)

    def gather(c, buf):
        return pltpu.make_async_copy(
            table_hbm.at[idx_v.at[pl.ds(pl.multiple_of(c * CH, 8), CH)]],
            rows_v.at[buf],
            in_sems.at[buf],
        )

    def put(c, buf):
        return pltpu.make_async_copy(
            rows_v.at[buf],
            out_hbm.at[b, pl.ds(s0 + c * CH, CH)],
            out_sems.at[buf],
        )

    gather(0, 0).start()
    gather(1, 1).start()
    gather(0, 0).wait()
    put(0, 0).start()

    @pl.loop(0, (NCHUNK - 2) // 2)
    def _(g):
        c1 = 2 * g + 1
        put(c1 - 1, 0).wait()
        gather(c1 + 1, 0).start()
        gather(c1, 1).wait()
        put(c1, 1).start()
        c2 = 2 * g + 2
        put(c2 - 1, 1).wait()
        gather(c2 + 1, 1).start()
        gather(c2, 0).wait()
        put(c2, 0).start()

    cl = NCHUNK - 1
    gather(cl, 1).wait()
    put(cl, 1).start()
    put(cl - 1, 0).wait()
    put(cl, 1).wait()


def kernel(input_ids, table):
    return _gather_call(input_ids.astype(jnp.int32), table)
